# Initial kernel scaffold; baseline (speedup 1.0000x reference)
import jax
import jax.numpy as jnp
from jax.experimental import pallas as pl


def kernel(x0, x1, x2, x3, conv_w, conv_b, obj_w, obj_b, bbx_w, bbx_b, valid_size):
    raise NotImplementedError("write your pallas kernel here")


# speedup vs baseline: 5.5566x; 5.5566x over previous
import functools
import math

import jax
import jax.numpy as jnp
import numpy as np
from jax.experimental import pallas as pl
from jax.experimental.pallas import tpu as pltpu

N_IMG = 2
C = 256
SCALES = [8.0]
RATIOS = [0.5, 1.0, 2.0]
STRIDES = [4, 8, 16, 32]
SHAPES = [(128, 128), (64, 64), (32, 32), (16, 16)]
TILE_H = [16, 16, 32, 16]
PRE_NMS = 6000
POST_NMS = 300
NMS_T = 0.7
NTOT = 3 * sum(h * w for h, w in SHAPES)
NPAD = 65536
ROWS = NPAD // 128
TOPROWS = 48
BIG = 1 << 30

_JS, _KS = [], []
_k = 2
while _k <= NPAD:
    _j = _k // 2
    while _j >= 1:
        _JS.append(_j)
        _KS.append(_k)
        _j //= 2
    _k *= 2
N_STAGES = len(_JS)


def _base_anchors(stride):
    out = []
    c = stride / 2.0
    for s in SCALES:
        for r in RATIOS:
            h = stride * s * math.sqrt(r)
            w = stride * s * math.sqrt(1.0 / r)
            out.append((np.float32(c - h / 2.0), np.float32(c - w / 2.0),
                        np.float32(c + h / 2.0), np.float32(c + w / 2.0)))
    return out


def _head_kernel(H, W, Th, stride, base, idx_base,
                 x0_ref, x1b_ref, x2b_ref, w9_ref, cb_ref, hw_ref, hb_ref,
                 vs_ref, s_ref, y1_ref, x1_ref, y2_ref, x2_ref, id_ref):
    n = pl.program_id(0)
    t = pl.program_id(1)
    N = Th * W
    xdx = [x0_ref, x1b_ref, x2b_ref]
    acc = jnp.zeros((C, N), jnp.float32)
    for dy in range(3):
        for dx in range(3):
            xs = xdx[dx][0, 0, :, dy * W:dy * W + N]
            acc = acc + jax.lax.dot(w9_ref[dy * 3 + dx], xs,
                                    preferred_element_type=jnp.float32)
    h1 = jnp.maximum(acc + cb_ref[...], 0.0)
    out16 = jax.lax.dot(hw_ref[...], h1,
                        preferred_element_type=jnp.float32) + hb_ref[...]

    hmax = vs_ref[n, 0].astype(jnp.float32)
    wmax = vs_ref[n, 1].astype(jnp.float32)
    lw = W.bit_length() - 1
    p = jax.lax.broadcasted_iota(jnp.int32, (1, N), 1)
    iy = p >> lw
    ix = p & (W - 1)
    gy = t * Th + iy
    gyf = (gy * stride).astype(jnp.float32)
    gxf = (ix * stride).astype(jnp.float32)
    flat3 = (gy * W + ix) * 3

    for a in range(3):
        s_ref[0, a:a + 1, :] = out16[a:a + 1, :]
        id_ref[0, a:a + 1, :] = idx_base + flat3 + a
        a0, a1, a2, a3 = base[a]
        A0 = a0 + gyf
        A1 = a1 + gxf
        A2 = a2 + gyf
        A3 = a3 + gxf
        ya = (A0 + A2) * 0.5
        xa = (A1 + A3) * 0.5
        ha = A2 - A0
        wa = A3 - A1
        d0 = out16[3 + 4 * a:4 + 4 * a, :]
        d1 = out16[4 + 4 * a:5 + 4 * a, :]
        d2 = out16[5 + 4 * a:6 + 4 * a, :]
        d3 = out16[6 + 4 * a:7 + 4 * a, :]
        yc = ya + d0 * ha
        xc = xa + d1 * wa
        hh = ha * jnp.exp(d2)
        ww = wa * jnp.exp(d3)
        b0 = yc - hh * 0.5
        b1 = xc - ww * 0.5
        b2 = yc + hh * 0.5
        b3 = xc + ww * 0.5
        b0 = jnp.minimum(jnp.maximum(b0, 0.0), hmax)
        b1 = jnp.minimum(jnp.maximum(b1, 0.0), wmax)
        b2 = jnp.minimum(jnp.maximum(b2, 0.0), hmax)
        b3 = jnp.minimum(jnp.maximum(b3, 0.0), wmax)
        y1_ref[0, a:a + 1, :] = b0
        x1_ref[0, a:a + 1, :] = b1
        y2_ref[0, a:a + 1, :] = b2
        x2_ref[0, a:a + 1, :] = b3


def _run_head(x, w9, cb, hw, hb, valid_size, level, idx_base):
    H, W = SHAPES[level]
    Th = TILE_H[level]
    T = H // Th
    stride = STRIDES[level]
    xp = jnp.pad(x, ((0, 0), (0, 0), (1, 1), (1, 1)))
    xbs = []
    for dx in range(3):
        xv = xp[:, :, :, dx:dx + W]
        xf = xv.reshape(N_IMG, C, (H + 2) * W)
        xbs.append(jnp.stack(
            [xf[:, :, t * Th * W:(t * Th + Th + 2) * W] for t in range(T)],
            axis=1))
    HW = H * W
    fs = jax.ShapeDtypeStruct((N_IMG, 3, HW), jnp.float32)
    out_shape = [fs, fs, fs, fs, fs,
                 jax.ShapeDtypeStruct((N_IMG, 3, HW), jnp.int32)]
    obs = pl.BlockSpec((1, 3, Th * W), lambda n, t: (n, 0, t))
    kern = functools.partial(_head_kernel, H, W, Th, stride,
                             _base_anchors(stride), idx_base)
    xspec = pl.BlockSpec((1, 1, C, (Th + 2) * W), lambda n, t: (n, t, 0, 0))
    return pl.pallas_call(
        kern,
        grid=(N_IMG, T),
        in_specs=[
            xspec, xspec, xspec,
            pl.BlockSpec((9, C, C), lambda n, t: (0, 0, 0)),
            pl.BlockSpec((C, 1), lambda n, t: (0, 0)),
            pl.BlockSpec((16, C), lambda n, t: (0, 0)),
            pl.BlockSpec((16, 1), lambda n, t: (0, 0)),
            pl.BlockSpec(memory_space=pltpu.SMEM),
        ],
        out_specs=[obs] * 6,
        out_shape=out_shape,
    )(xbs[0], xbs[1], xbs[2], w9, cb, hw, hb, valid_size)


def _sort_nms_kernel(js_ref, ks_ref, s_ref, i_ref, y1_ref, x1_ref, y2_ref,
                     x2_ref, o_ref, ksc, isc, asc, bsc, csc, dsc):
    gio = (jax.lax.broadcasted_iota(jnp.int32, (ROWS, 128), 0) * 128
           + jax.lax.broadcasted_iota(jnp.int32, (ROWS, 128), 1))
    ksc[...] = s_ref[...]
    isc[...] = i_ref[...]
    asc[...] = y1_ref[...]
    bsc[...] = x1_ref[...]
    csc[...] = y2_ref[...]
    dsc[...] = x2_ref[...]
    scr = (ksc, isc, asc, bsc, csc, dsc)

    def stage(s, _):
        j = js_ref[s]
        kstep = ks_ref[s]
        sh1 = j >> 7
        sh2 = j & 127
        m = ((gio & j) == 0)[None]

        def part(x):
            x1r = jnp.where(m, pltpu.roll(x, (ROWS - sh1) & (ROWS - 1), 1),
                            pltpu.roll(x, sh1, 1))
            return jnp.where(m, pltpu.roll(x1r, (128 - sh2) & 127, 2),
                             pltpu.roll(x1r, sh2, 2))

        kk = ksc[...]
        ii = isc[...]
        vals = [r[...] for r in scr[2:]]
        kp = part(kk)
        ip = part(ii)
        want_first = (m == ((gio & kstep) == 0)[None])
        cur_first = (kk > kp) | ((kk == kp) & (ii < ip))
        tc = want_first == cur_first
        ksc[...] = jnp.where(tc, kk, kp)
        isc[...] = jnp.where(tc, ii, ip)
        for r, v in zip(scr[2:], vals):
            r[...] = jnp.where(tc, v, part(v))
        return 0

    jax.lax.fori_loop(0, N_STAGES, stage, 0)
    v1 = asc[...]
    u1 = bsc[...]
    v2 = csc[...]
    u2 = dsc[...]

    y1s = v1[:, :TOPROWS, :]
    x1s = u1[:, :TOPROWS, :]
    y2s = v2[:, :TOPROWS, :]
    x2s = u2[:, :TOPROWS, :]
    g48 = gio[:TOPROWS][None]
    areas = (y2s - y1s) * (x2s - x1s)
    active0 = jnp.broadcast_to(g48 < PRE_NMS,
                               (N_IMG, TOPROWS, 128)).astype(jnp.int32)
    li = jax.lax.broadcasted_iota(jnp.int32, (1, 1, 128), 2)

    def body(i, active_i):
        active = active_i != 0
        mm = jnp.where(active, g48, BIG)
        mn = jnp.min(mm, axis=(1, 2), keepdims=True)
        valid = mn < BIG
        oh = (g48 == mn).astype(jnp.float32)
        y1c = jnp.sum(y1s * oh, axis=(1, 2), keepdims=True)
        x1c = jnp.sum(x1s * oh, axis=(1, 2), keepdims=True)
        y2c = jnp.sum(y2s * oh, axis=(1, 2), keepdims=True)
        x2c = jnp.sum(x2s * oh, axis=(1, 2), keepdims=True)
        ac = (y2c - y1c) * (x2c - x1c)
        yy1 = jnp.maximum(y1c, y1s)
        xx1 = jnp.maximum(x1c, x1s)
        yy2 = jnp.minimum(y2c, y2s)
        xx2 = jnp.minimum(x2c, x2s)
        inter = jnp.maximum(yy2 - yy1, 0.0) * jnp.maximum(xx2 - xx1, 0.0)
        iou = inter / (ac + areas - inter + 1e-9)
        sup = (iou > NMS_T) & valid
        active = active & (~sup) & (g48 != mn)
        row = (jnp.where(li == 0, y1c, 0.0) + jnp.where(li == 1, x1c, 0.0)
               + jnp.where(li == 2, y2c, 0.0) + jnp.where(li == 3, x2c, 0.0))
        row = jnp.where(valid, row, 0.0)
        o_ref[:, pl.ds(i, 1), :] = row
        return active.astype(jnp.int32)

    jax.lax.fori_loop(0, POST_NMS, body, active0)


def kernel(x0, x1, x2, x3, conv_w, conv_b, obj_w, obj_b, bbx_w, bbx_b,
           valid_size):
    xs = [x0, x1, x2, x3]
    w9 = jnp.transpose(conv_w, (2, 3, 0, 1)).reshape(9, C, C)
    cb = conv_b.reshape(C, 1)
    hw = jnp.concatenate([obj_w[:, :, 0, 0], bbx_w[:, :, 0, 0],
                          jnp.zeros((1, C), jnp.float32)], axis=0)
    hb = jnp.concatenate([obj_b, bbx_b,
                          jnp.zeros((1,), jnp.float32)]).reshape(16, 1)

    parts = [[] for _ in range(6)]
    idx_base = 0
    for level in range(4):
        outs = _run_head(xs[level], w9, cb, hw, hb, valid_size, level,
                         idx_base)
        for p, o in zip(parts, outs):
            p.append(o.reshape(N_IMG, -1))
        idx_base += 3 * SHAPES[level][0] * SHAPES[level][1]

    npad = NPAD - NTOT
    sc = jnp.concatenate(parts[0] + [jnp.full((N_IMG, npad), -jnp.inf,
                                              jnp.float32)], axis=1)
    y1 = jnp.concatenate(parts[1] + [jnp.zeros((N_IMG, npad), jnp.float32)],
                         axis=1)
    x1 = jnp.concatenate(parts[2] + [jnp.zeros((N_IMG, npad), jnp.float32)],
                         axis=1)
    y2 = jnp.concatenate(parts[3] + [jnp.zeros((N_IMG, npad), jnp.float32)],
                         axis=1)
    x2 = jnp.concatenate(parts[4] + [jnp.zeros((N_IMG, npad), jnp.float32)],
                         axis=1)
    ids = jnp.concatenate(
        parts[5] + [jnp.broadcast_to(jnp.arange(NTOT, NPAD, dtype=jnp.int32),
                                     (N_IMG, npad))], axis=1)

    def r(a):
        return a.reshape(N_IMG, ROWS, 128)

    js = jnp.asarray(np.array(_JS, np.int32))
    ks = jnp.asarray(np.array(_KS, np.int32))
    props = pl.pallas_call(
        _sort_nms_kernel,
        in_specs=[pl.BlockSpec(memory_space=pltpu.SMEM)] * 2
        + [pl.BlockSpec((N_IMG, ROWS, 128), lambda: (0, 0, 0))] * 6,
        out_specs=pl.BlockSpec((N_IMG, 304, 128), lambda: (0, 0, 0)),
        out_shape=jax.ShapeDtypeStruct((N_IMG, 304, 128), jnp.float32),
        scratch_shapes=[pltpu.VMEM((N_IMG, ROWS, 128), jnp.float32),
                        pltpu.VMEM((N_IMG, ROWS, 128), jnp.int32)]
        + [pltpu.VMEM((N_IMG, ROWS, 128), jnp.float32)] * 4,
    )(js, ks, r(sc), r(ids), r(y1), r(x1), r(y2), r(x2))
    return props[:, :POST_NMS, :4]


# speedup vs baseline: 8.2203x; 1.4794x over previous
import functools
import math

import jax
import jax.numpy as jnp
import numpy as np
from jax import lax
from jax.experimental import pallas as pl
from jax.experimental.pallas import tpu as pltpu
from jax.experimental.pallas import tpu_sc as plsc

N_IMG = 2
C = 256
SCALES = [8.0]
RATIOS = [0.5, 1.0, 2.0]
STRIDES = [4, 8, 16, 32]
SHAPES = [(128, 128), (64, 64), (32, 32), (16, 16)]
TILE_H = [16, 16, 32, 16]
PRE_NMS = 6000
POST_NMS = 300
NMS_T = 0.7
NTOT = 3 * sum(h * w for h, w in SHAPES)
NPAD = 65536
ROWS = NPAD // 128
TOPROWS = 48
BIG = 1 << 30

_JS, _KS = [], []
_k = 2
while _k <= NPAD:
    _j = _k // 2
    while _j >= 1:
        _JS.append(_j)
        _KS.append(_k)
        _j //= 2
    _k *= 2
N_STAGES = len(_JS)

_PERM = np.zeros(NPAD, np.int32)
_b = 0
for _h, _w in SHAPES:
    _hw = _h * _w
    _yx = np.arange(_hw)
    for _a in range(3):
        _PERM[_b + _yx * 3 + _a] = _b + _a * _hw + _yx
    _b += 3 * _hw
_PERM[NTOT:] = np.arange(NTOT, NPAD)
_PERM_FULL = np.concatenate(
    [_PERM + _i * NPAD for _i in range(N_IMG)]).astype(np.int32)


def _base_anchors(stride):
    out = []
    c = stride / 2.0
    for s in SCALES:
        for r in RATIOS:
            h = stride * s * math.sqrt(r)
            w = stride * s * math.sqrt(1.0 / r)
            out.append((np.float32(c - h / 2.0), np.float32(c - w / 2.0),
                        np.float32(c + h / 2.0), np.float32(c + w / 2.0)))
    return out


def _head_kernel(H, W, Th, stride, base, idx_base,
                 x0_ref, x1b_ref, x2b_ref, w9_ref, cb_ref, hw_ref, hb_ref,
                 vs_ref, s_ref, y1_ref, x1_ref, y2_ref, x2_ref, id_ref):
    n = pl.program_id(0)
    t = pl.program_id(1)
    N = Th * W
    xdx = [x0_ref, x1b_ref, x2b_ref]
    acc = jnp.zeros((C, N), jnp.float32)
    for dy in range(3):
        for dx in range(3):
            xs = xdx[dx][0, 0, :, dy * W:dy * W + N]
            acc = acc + jax.lax.dot(w9_ref[dy * 3 + dx], xs,
                                    preferred_element_type=jnp.float32)
    h1 = jnp.maximum(acc + cb_ref[...], 0.0)
    out16 = jax.lax.dot(hw_ref[...], h1,
                        preferred_element_type=jnp.float32) + hb_ref[...]

    hmax = vs_ref[n, 0].astype(jnp.float32)
    wmax = vs_ref[n, 1].astype(jnp.float32)
    lw = W.bit_length() - 1
    p = jax.lax.broadcasted_iota(jnp.int32, (1, N), 1)
    iy = p >> lw
    ix = p & (W - 1)
    gy = t * Th + iy
    gyf = (gy * stride).astype(jnp.float32)
    gxf = (ix * stride).astype(jnp.float32)
    flat3 = (gy * W + ix) * 3

    for a in range(3):
        s_ref[0, a:a + 1, :] = out16[a:a + 1, :]
        id_ref[0, a:a + 1, :] = idx_base + flat3 + a
        a0, a1, a2, a3 = base[a]
        A0 = a0 + gyf
        A1 = a1 + gxf
        A2 = a2 + gyf
        A3 = a3 + gxf
        ya = (A0 + A2) * 0.5
        xa = (A1 + A3) * 0.5
        ha = A2 - A0
        wa = A3 - A1
        d0 = out16[3 + 4 * a:4 + 4 * a, :]
        d1 = out16[4 + 4 * a:5 + 4 * a, :]
        d2 = out16[5 + 4 * a:6 + 4 * a, :]
        d3 = out16[6 + 4 * a:7 + 4 * a, :]
        yc = ya + d0 * ha
        xc = xa + d1 * wa
        hh = ha * jnp.exp(d2)
        ww = wa * jnp.exp(d3)
        b0 = yc - hh * 0.5
        b1 = xc - ww * 0.5
        b2 = yc + hh * 0.5
        b3 = xc + ww * 0.5
        b0 = jnp.minimum(jnp.maximum(b0, 0.0), hmax)
        b1 = jnp.minimum(jnp.maximum(b1, 0.0), wmax)
        b2 = jnp.minimum(jnp.maximum(b2, 0.0), hmax)
        b3 = jnp.minimum(jnp.maximum(b3, 0.0), wmax)
        y1_ref[0, a:a + 1, :] = b0
        x1_ref[0, a:a + 1, :] = b1
        y2_ref[0, a:a + 1, :] = b2
        x2_ref[0, a:a + 1, :] = b3


def _run_head(x, w9, cb, hw, hb, valid_size, level, idx_base):
    H, W = SHAPES[level]
    Th = TILE_H[level]
    T = H // Th
    stride = STRIDES[level]
    xp = jnp.pad(x, ((0, 0), (0, 0), (1, 1), (1, 1)))
    xbs = []
    for dx in range(3):
        xv = xp[:, :, :, dx:dx + W]
        xf = xv.reshape(N_IMG, C, (H + 2) * W)
        xbs.append(jnp.stack(
            [xf[:, :, t * Th * W:(t * Th + Th + 2) * W] for t in range(T)],
            axis=1))
    HW = H * W
    fs = jax.ShapeDtypeStruct((N_IMG, 3, HW), jnp.float32)
    out_shape = [fs, fs, fs, fs, fs,
                 jax.ShapeDtypeStruct((N_IMG, 3, HW), jnp.int32)]
    obs = pl.BlockSpec((1, 3, Th * W), lambda n, t: (n, 0, t))
    kern = functools.partial(_head_kernel, H, W, Th, stride,
                             _base_anchors(stride), idx_base)
    xspec = pl.BlockSpec((1, 1, C, (Th + 2) * W), lambda n, t: (n, t, 0, 0))
    return pl.pallas_call(
        kern,
        grid=(N_IMG, T),
        in_specs=[
            xspec, xspec, xspec,
            pl.BlockSpec((9, C, C), lambda n, t: (0, 0, 0)),
            pl.BlockSpec((C, 1), lambda n, t: (0, 0)),
            pl.BlockSpec((16, C), lambda n, t: (0, 0)),
            pl.BlockSpec((16, 1), lambda n, t: (0, 0)),
            pl.BlockSpec(memory_space=pltpu.SMEM),
        ],
        out_specs=[obs] * 6,
        out_shape=out_shape,
    )(xbs[0], xbs[1], xbs[2], w9, cb, hw, hb, valid_size)


def _sort_kernel(js_ref, ks_ref, s_ref, i_ref, o_ref, ksc, isc):
    gio = (jax.lax.broadcasted_iota(jnp.int32, (ROWS, 128), 0) * 128
           + jax.lax.broadcasted_iota(jnp.int32, (ROWS, 128), 1))
    ksc[...] = s_ref[...]
    isc[...] = i_ref[...]

    def stage(s, _):
        j = js_ref[s]
        kstep = ks_ref[s]
        sh1 = j >> 7
        sh2 = j & 127
        m = ((gio & j) == 0)[None]

        def part(x):
            x1r = jnp.where(m, pltpu.roll(x, (ROWS - sh1) & (ROWS - 1), 1),
                            pltpu.roll(x, sh1, 1))
            return jnp.where(m, pltpu.roll(x1r, (128 - sh2) & 127, 2),
                             pltpu.roll(x1r, sh2, 2))

        kk = ksc[...]
        ii = isc[...]
        kp = part(kk)
        ip = part(ii)
        want_first = (m == ((gio & kstep) == 0)[None])
        cur_first = (kk > kp) | ((kk == kp) & (ii < ip))
        tc = want_first == cur_first
        ksc[...] = jnp.where(tc, kk, kp)
        isc[...] = jnp.where(tc, ii, ip)
        return 0

    jax.lax.fori_loop(0, N_STAGES, stage, 0)
    o_ref[...] = isc[:, :TOPROWS, :]


def _make_gather():
    NC, NS = 2, 16
    NW = NC * NS
    B = N_IMG * TOPROWS * 128
    b_per_w = B // NW
    mesh = plsc.VectorSubcoreMesh(core_axis_name="c", subcore_axis_name="s")

    fdt = jax.ShapeDtypeStruct((B,), jnp.float32)

    @functools.partial(
        pl.kernel, mesh=mesh,
        out_type=[fdt, fdt, fdt, fdt],
        scratch_types=[
            pltpu.VMEM((b_per_w,), jnp.int32),
            pltpu.VMEM((b_per_w,), jnp.int32),
            pltpu.VMEM((b_per_w,), jnp.float32),
            pltpu.VMEM((b_per_w,), jnp.float32),
            pltpu.VMEM((b_per_w,), jnp.float32),
            pltpu.VMEM((b_per_w,), jnp.float32),
            pltpu.SemaphoreType.DMA,
        ],
    )
    def gk(c0_hbm, c1_hbm, c2_hbm, c3_hbm, perm_hbm, idx_hbm,
           o0, o1, o2, o3, idx_v, pos_v, b0, b1, b2, b3, sem):
        wid = lax.axis_index("s") * NC + lax.axis_index("c")
        base = wid * b_per_w
        pltpu.sync_copy(idx_hbm.at[pl.ds(base, b_per_w)], idx_v)
        pltpu.async_copy(perm_hbm.at[idx_v], pos_v, sem).wait()
        for src, buf, out in ((c0_hbm, b0, o0), (c1_hbm, b1, o1),
                              (c2_hbm, b2, o2), (c3_hbm, b3, o3)):
            pltpu.async_copy(src.at[pos_v], buf, sem).wait()
            pltpu.sync_copy(buf, out.at[pl.ds(base, b_per_w)])

    return gk


def _nms_kernel(y1_ref, x1_ref, y2_ref, x2_ref, o_ref):
    y1s = y1_ref[...]
    x1s = x1_ref[...]
    y2s = y2_ref[...]
    x2s = x2_ref[...]
    g48 = (jax.lax.broadcasted_iota(jnp.int32, (TOPROWS, 128), 0) * 128
           + jax.lax.broadcasted_iota(jnp.int32, (TOPROWS, 128), 1))[None]
    areas = (y2s - y1s) * (x2s - x1s)
    active0 = jnp.broadcast_to(g48 < PRE_NMS,
                               (N_IMG, TOPROWS, 128)).astype(jnp.int32)
    li = jax.lax.broadcasted_iota(jnp.int32, (1, 1, 128), 2)

    def body(i, active_i):
        active = active_i != 0
        mm = jnp.where(active, g48, BIG)
        mn = jnp.min(mm, axis=(1, 2), keepdims=True)
        valid = mn < BIG
        oh = (g48 == mn).astype(jnp.float32)
        y1c = jnp.sum(y1s * oh, axis=(1, 2), keepdims=True)
        x1c = jnp.sum(x1s * oh, axis=(1, 2), keepdims=True)
        y2c = jnp.sum(y2s * oh, axis=(1, 2), keepdims=True)
        x2c = jnp.sum(x2s * oh, axis=(1, 2), keepdims=True)
        ac = (y2c - y1c) * (x2c - x1c)
        yy1 = jnp.maximum(y1c, y1s)
        xx1 = jnp.maximum(x1c, x1s)
        yy2 = jnp.minimum(y2c, y2s)
        xx2 = jnp.minimum(x2c, x2s)
        inter = jnp.maximum(yy2 - yy1, 0.0) * jnp.maximum(xx2 - xx1, 0.0)
        iou = inter / (ac + areas - inter + 1e-9)
        sup = (iou > NMS_T) & valid
        active = active & (~sup) & (g48 != mn)
        row = (jnp.where(li == 0, y1c, 0.0) + jnp.where(li == 1, x1c, 0.0)
               + jnp.where(li == 2, y2c, 0.0) + jnp.where(li == 3, x2c, 0.0))
        row = jnp.where(valid, row, 0.0)
        o_ref[:, pl.ds(i, 1), :] = row
        return active.astype(jnp.int32)

    jax.lax.fori_loop(0, POST_NMS, body, active0)


def kernel(x0, x1, x2, x3, conv_w, conv_b, obj_w, obj_b, bbx_w, bbx_b,
           valid_size):
    xs = [x0, x1, x2, x3]
    w9 = jnp.transpose(conv_w, (2, 3, 0, 1)).reshape(9, C, C)
    cb = conv_b.reshape(C, 1)
    hw = jnp.concatenate([obj_w[:, :, 0, 0], bbx_w[:, :, 0, 0],
                          jnp.zeros((1, C), jnp.float32)], axis=0)
    hb = jnp.concatenate([obj_b, bbx_b,
                          jnp.zeros((1,), jnp.float32)]).reshape(16, 1)

    parts = [[] for _ in range(6)]
    idx_base = 0
    for level in range(4):
        outs = _run_head(xs[level], w9, cb, hw, hb, valid_size, level,
                         idx_base)
        for p, o in zip(parts, outs):
            p.append(o.reshape(N_IMG, -1))
        idx_base += 3 * SHAPES[level][0] * SHAPES[level][1]

    npad = NPAD - NTOT
    sc = jnp.concatenate(parts[0] + [jnp.full((N_IMG, npad), -jnp.inf,
                                              jnp.float32)], axis=1)
    ids = jnp.concatenate(
        parts[5] + [jnp.broadcast_to(jnp.arange(NTOT, NPAD, dtype=jnp.int32),
                                     (N_IMG, npad))], axis=1)

    def r(a):
        return a.reshape(N_IMG, ROWS, 128)

    js = jnp.asarray(np.array(_JS, np.int32))
    ks = jnp.asarray(np.array(_KS, np.int32))
    topidx = pl.pallas_call(
        _sort_kernel,
        in_specs=[pl.BlockSpec(memory_space=pltpu.SMEM)] * 2
        + [pl.BlockSpec((N_IMG, ROWS, 128), lambda: (0, 0, 0))] * 2,
        out_specs=pl.BlockSpec((N_IMG, TOPROWS, 128), lambda: (0, 0, 0)),
        out_shape=jax.ShapeDtypeStruct((N_IMG, TOPROWS, 128), jnp.int32),
        scratch_shapes=[pltpu.VMEM((N_IMG, ROWS, 128), jnp.float32),
                        pltpu.VMEM((N_IMG, ROWS, 128), jnp.int32)],
    )(js, ks, r(sc), r(ids))

    planes = [jnp.pad(jnp.concatenate(parts[i], axis=1),
                      ((0, 0), (0, npad))).reshape(N_IMG * NPAD)
              for i in (1, 2, 3, 4)]
    perm = jnp.asarray(_PERM_FULL)
    idxb = (topidx.reshape(N_IMG, TOPROWS * 128)
            + (jnp.arange(N_IMG, dtype=jnp.int32) * NPAD)[:, None]
            ).reshape(N_IMG * TOPROWS * 128)
    g0, g1, g2, g3 = _make_gather()(planes[0], planes[1], planes[2],
                                    planes[3], perm, idxb)

    def q(a):
        return a.reshape(N_IMG, TOPROWS, 128)

    props = pl.pallas_call(
        _nms_kernel,
        in_specs=[pl.BlockSpec((N_IMG, TOPROWS, 128), lambda: (0, 0, 0))] * 4,
        out_specs=pl.BlockSpec((N_IMG, 304, 128), lambda: (0, 0, 0)),
        out_shape=jax.ShapeDtypeStruct((N_IMG, 304, 128), jnp.float32),
    )(q(g0), q(g1), q(g2), q(g3))
    return props[:, :POST_NMS, :4]


# speedup vs baseline: 8.6240x; 1.0491x over previous
import functools
import math

import jax
import jax.numpy as jnp
import numpy as np
from jax import lax
from jax.experimental import pallas as pl
from jax.experimental.pallas import tpu as pltpu
from jax.experimental.pallas import tpu_sc as plsc

N_IMG = 2
C = 256
SCALES = [8.0]
RATIOS = [0.5, 1.0, 2.0]
STRIDES = [4, 8, 16, 32]
SHAPES = [(128, 128), (64, 64), (32, 32), (16, 16)]
TILE_H = [32, 32, 32, 16]
PRE_NMS = 6000
POST_NMS = 300
NMS_T = 0.7
NTOT = 3 * sum(h * w for h, w in SHAPES)
NPAD = 65536
ROWS = NPAD // 128
TOPROWS = 48
BIG = 1 << 30

_JS, _KS = [], []
_k = 2
while _k <= NPAD:
    _j = _k // 2
    while _j >= 1:
        _JS.append(_j)
        _KS.append(_k)
        _j //= 2
    _k *= 2
N_STAGES = len(_JS)

_PERM = np.zeros(NPAD, np.int32)
_b = 0
for _h, _w in SHAPES:
    _hw = _h * _w
    _yx = np.arange(_hw)
    for _a in range(3):
        _PERM[_b + _yx * 3 + _a] = _b + _a * _hw + _yx
    _b += 3 * _hw
_PERM[NTOT:] = np.arange(NTOT, NPAD)
_PERM_FULL = np.concatenate(
    [_PERM + _i * NPAD for _i in range(N_IMG)]).astype(np.int32)


def _base_anchors(stride):
    out = []
    c = stride / 2.0
    for s in SCALES:
        for r in RATIOS:
            h = stride * s * math.sqrt(r)
            w = stride * s * math.sqrt(1.0 / r)
            out.append((np.float32(c - h / 2.0), np.float32(c - w / 2.0),
                        np.float32(c + h / 2.0), np.float32(c + w / 2.0)))
    return out


def _head_kernel(H, W, Th, stride, base, idx_base,
                 x0_ref, x1b_ref, x2b_ref, w9_ref, cb_ref, hw_ref, hb_ref,
                 vs_ref, s_ref, y1_ref, x1_ref, y2_ref, x2_ref, id_ref):
    n = pl.program_id(0)
    t = pl.program_id(1)
    N = Th * W
    xdx = [x0_ref, x1b_ref, x2b_ref]
    acc = jnp.zeros((C, N), jnp.float32)
    for dy in range(3):
        for dx in range(3):
            xs = xdx[dx][0, 0, :, dy * W:dy * W + N]
            acc = acc + jax.lax.dot(w9_ref[dy * 3 + dx], xs,
                                    preferred_element_type=jnp.float32)
    h1 = jnp.maximum(acc + cb_ref[...], 0.0)
    out16 = jax.lax.dot(hw_ref[...], h1,
                        preferred_element_type=jnp.float32) + hb_ref[...]

    hmax = vs_ref[n, 0].astype(jnp.float32)
    wmax = vs_ref[n, 1].astype(jnp.float32)
    lw = W.bit_length() - 1
    p = jax.lax.broadcasted_iota(jnp.int32, (1, N), 1)
    iy = p >> lw
    ix = p & (W - 1)
    gy = t * Th + iy
    gyf = (gy * stride).astype(jnp.float32)
    gxf = (ix * stride).astype(jnp.float32)
    flat3 = (gy * W + ix) * 3

    for a in range(3):
        s_ref[0, a:a + 1, :] = out16[a:a + 1, :]
        id_ref[0, a:a + 1, :] = idx_base + flat3 + a
        a0, a1, a2, a3 = base[a]
        A0 = a0 + gyf
        A1 = a1 + gxf
        A2 = a2 + gyf
        A3 = a3 + gxf
        ya = (A0 + A2) * 0.5
        xa = (A1 + A3) * 0.5
        ha = A2 - A0
        wa = A3 - A1
        d0 = out16[3 + 4 * a:4 + 4 * a, :]
        d1 = out16[4 + 4 * a:5 + 4 * a, :]
        d2 = out16[5 + 4 * a:6 + 4 * a, :]
        d3 = out16[6 + 4 * a:7 + 4 * a, :]
        yc = ya + d0 * ha
        xc = xa + d1 * wa
        hh = ha * jnp.exp(d2)
        ww = wa * jnp.exp(d3)
        b0 = yc - hh * 0.5
        b1 = xc - ww * 0.5
        b2 = yc + hh * 0.5
        b3 = xc + ww * 0.5
        b0 = jnp.minimum(jnp.maximum(b0, 0.0), hmax)
        b1 = jnp.minimum(jnp.maximum(b1, 0.0), wmax)
        b2 = jnp.minimum(jnp.maximum(b2, 0.0), hmax)
        b3 = jnp.minimum(jnp.maximum(b3, 0.0), wmax)
        y1_ref[0, a:a + 1, :] = b0
        x1_ref[0, a:a + 1, :] = b1
        y2_ref[0, a:a + 1, :] = b2
        x2_ref[0, a:a + 1, :] = b3


def _run_head(x, w9, cb, hw, hb, valid_size, level, idx_base):
    H, W = SHAPES[level]
    Th = TILE_H[level]
    T = H // Th
    stride = STRIDES[level]
    xp = jnp.pad(x, ((0, 0), (0, 0), (1, 1), (1, 1)))
    xbs = []
    for dx in range(3):
        xv = xp[:, :, :, dx:dx + W]
        xf = xv.reshape(N_IMG, C, (H + 2) * W)
        xbs.append(jnp.stack(
            [xf[:, :, t * Th * W:(t * Th + Th + 2) * W] for t in range(T)],
            axis=1))
    HW = H * W
    fs = jax.ShapeDtypeStruct((N_IMG, 3, HW), jnp.float32)
    out_shape = [fs, fs, fs, fs, fs,
                 jax.ShapeDtypeStruct((N_IMG, 3, HW), jnp.int32)]
    obs = pl.BlockSpec((1, 3, Th * W), lambda n, t: (n, 0, t))
    kern = functools.partial(_head_kernel, H, W, Th, stride,
                             _base_anchors(stride), idx_base)
    xspec = pl.BlockSpec((1, 1, C, (Th + 2) * W), lambda n, t: (n, t, 0, 0))
    return pl.pallas_call(
        kern,
        grid=(N_IMG, T),
        in_specs=[
            xspec, xspec, xspec,
            pl.BlockSpec((9, C, C), lambda n, t: (0, 0, 0)),
            pl.BlockSpec((C, 1), lambda n, t: (0, 0)),
            pl.BlockSpec((16, C), lambda n, t: (0, 0)),
            pl.BlockSpec((16, 1), lambda n, t: (0, 0)),
            pl.BlockSpec(memory_space=pltpu.SMEM),
        ],
        out_specs=[obs] * 6,
        out_shape=out_shape,
    )(xbs[0], xbs[1], xbs[2], w9, cb, hw, hb, valid_size)


def _sort_kernel(js_ref, ks_ref, s_ref, i_ref, o_ref, ksc, isc):
    gio = (jax.lax.broadcasted_iota(jnp.int32, (ROWS, 128), 0) * 128
           + jax.lax.broadcasted_iota(jnp.int32, (ROWS, 128), 1))
    ksc[...] = s_ref[...]
    isc[...] = i_ref[...]

    def stage(s, _):
        j = js_ref[s]
        kstep = ks_ref[s]
        sh1 = j >> 7
        sh2 = j & 127
        m = ((gio & j) == 0)[None]

        def part(x):
            x1r = jnp.where(m, pltpu.roll(x, (ROWS - sh1) & (ROWS - 1), 1),
                            pltpu.roll(x, sh1, 1))
            return jnp.where(m, pltpu.roll(x1r, (128 - sh2) & 127, 2),
                             pltpu.roll(x1r, sh2, 2))

        kk = ksc[...]
        ii = isc[...]
        kp = part(kk)
        ip = part(ii)
        want_first = (m == ((gio & kstep) == 0)[None])
        cur_first = (kk > kp) | ((kk == kp) & (ii < ip))
        tc = want_first == cur_first
        ksc[...] = jnp.where(tc, kk, kp)
        isc[...] = jnp.where(tc, ii, ip)
        return 0

    jax.lax.fori_loop(0, N_STAGES, stage, 0)
    o_ref[...] = isc[:, :TOPROWS, :]


def _make_gather():
    NC, NS = 2, 16
    NW = NC * NS
    B = N_IMG * TOPROWS * 128
    b_per_w = B // NW
    mesh = plsc.VectorSubcoreMesh(core_axis_name="c", subcore_axis_name="s")

    fdt = jax.ShapeDtypeStruct((B,), jnp.float32)

    @functools.partial(
        pl.kernel, mesh=mesh,
        out_type=[fdt, fdt, fdt, fdt],
        scratch_types=[
            pltpu.VMEM((b_per_w,), jnp.int32),
            pltpu.VMEM((b_per_w,), jnp.int32),
            pltpu.VMEM((b_per_w,), jnp.float32),
            pltpu.VMEM((b_per_w,), jnp.float32),
            pltpu.VMEM((b_per_w,), jnp.float32),
            pltpu.VMEM((b_per_w,), jnp.float32),
            pltpu.SemaphoreType.DMA,
        ],
    )
    def gk(c0_hbm, c1_hbm, c2_hbm, c3_hbm, perm_hbm, idx_hbm,
           o0, o1, o2, o3, idx_v, pos_v, b0, b1, b2, b3, sem):
        wid = lax.axis_index("s") * NC + lax.axis_index("c")
        base = wid * b_per_w
        pltpu.sync_copy(idx_hbm.at[pl.ds(base, b_per_w)], idx_v)
        pltpu.async_copy(perm_hbm.at[idx_v], pos_v, sem).wait()
        for src, buf, out in ((c0_hbm, b0, o0), (c1_hbm, b1, o1),
                              (c2_hbm, b2, o2), (c3_hbm, b3, o3)):
            pltpu.async_copy(src.at[pos_v], buf, sem).wait()
            pltpu.sync_copy(buf, out.at[pl.ds(base, b_per_w)])

    return gk


def _nms_kernel(y1_ref, x1_ref, y2_ref, x2_ref, o_ref):
    y1s = y1_ref[...]
    x1s = x1_ref[...]
    y2s = y2_ref[...]
    x2s = x2_ref[...]
    g48 = (jax.lax.broadcasted_iota(jnp.int32, (TOPROWS, 128), 0) * 128
           + jax.lax.broadcasted_iota(jnp.int32, (TOPROWS, 128), 1))[None]
    areas = (y2s - y1s) * (x2s - x1s)
    active0 = jnp.broadcast_to(g48 < PRE_NMS,
                               (N_IMG, TOPROWS, 128)).astype(jnp.int32)
    li = jax.lax.broadcasted_iota(jnp.int32, (1, 1, 128), 2)

    def body(i, active_i):
        active = active_i != 0
        mm = jnp.where(active, g48, BIG)
        mn = jnp.min(mm, axis=(1, 2), keepdims=True)
        valid = mn < BIG
        oh = (g48 == mn).astype(jnp.float32)
        y1c = jnp.sum(y1s * oh, axis=(1, 2), keepdims=True)
        x1c = jnp.sum(x1s * oh, axis=(1, 2), keepdims=True)
        y2c = jnp.sum(y2s * oh, axis=(1, 2), keepdims=True)
        x2c = jnp.sum(x2s * oh, axis=(1, 2), keepdims=True)
        ac = (y2c - y1c) * (x2c - x1c)
        yy1 = jnp.maximum(y1c, y1s)
        xx1 = jnp.maximum(x1c, x1s)
        yy2 = jnp.minimum(y2c, y2s)
        xx2 = jnp.minimum(x2c, x2s)
        inter = jnp.maximum(yy2 - yy1, 0.0) * jnp.maximum(xx2 - xx1, 0.0)
        iou = inter / (ac + areas - inter + 1e-9)
        sup = (iou > NMS_T) & valid
        active = active & (~sup) & (g48 != mn)
        row = (jnp.where(li == 0, y1c, 0.0) + jnp.where(li == 1, x1c, 0.0)
               + jnp.where(li == 2, y2c, 0.0) + jnp.where(li == 3, x2c, 0.0))
        row = jnp.where(valid, row, 0.0)
        o_ref[:, pl.ds(i, 1), :] = row
        return active.astype(jnp.int32)

    jax.lax.fori_loop(0, POST_NMS, body, active0)


def kernel(x0, x1, x2, x3, conv_w, conv_b, obj_w, obj_b, bbx_w, bbx_b,
           valid_size):
    xs = [x0, x1, x2, x3]
    w9 = jnp.transpose(conv_w, (2, 3, 0, 1)).reshape(9, C, C)
    cb = conv_b.reshape(C, 1)
    hw = jnp.concatenate([obj_w[:, :, 0, 0], bbx_w[:, :, 0, 0],
                          jnp.zeros((1, C), jnp.float32)], axis=0)
    hb = jnp.concatenate([obj_b, bbx_b,
                          jnp.zeros((1,), jnp.float32)]).reshape(16, 1)

    parts = [[] for _ in range(6)]
    idx_base = 0
    for level in range(4):
        outs = _run_head(xs[level], w9, cb, hw, hb, valid_size, level,
                         idx_base)
        for p, o in zip(parts, outs):
            p.append(o.reshape(N_IMG, -1))
        idx_base += 3 * SHAPES[level][0] * SHAPES[level][1]

    npad = NPAD - NTOT
    sc = jnp.concatenate(parts[0] + [jnp.full((N_IMG, npad), -jnp.inf,
                                              jnp.float32)], axis=1)
    ids = jnp.concatenate(
        parts[5] + [jnp.broadcast_to(jnp.arange(NTOT, NPAD, dtype=jnp.int32),
                                     (N_IMG, npad))], axis=1)

    def r(a):
        return a.reshape(N_IMG, ROWS, 128)

    js = jnp.asarray(np.array(_JS, np.int32))
    ks = jnp.asarray(np.array(_KS, np.int32))
    topidx = pl.pallas_call(
        _sort_kernel,
        in_specs=[pl.BlockSpec(memory_space=pltpu.SMEM)] * 2
        + [pl.BlockSpec((N_IMG, ROWS, 128), lambda: (0, 0, 0))] * 2,
        out_specs=pl.BlockSpec((N_IMG, TOPROWS, 128), lambda: (0, 0, 0)),
        out_shape=jax.ShapeDtypeStruct((N_IMG, TOPROWS, 128), jnp.int32),
        scratch_shapes=[pltpu.VMEM((N_IMG, ROWS, 128), jnp.float32),
                        pltpu.VMEM((N_IMG, ROWS, 128), jnp.int32)],
    )(js, ks, r(sc), r(ids))

    planes = [jnp.pad(jnp.concatenate(parts[i], axis=1),
                      ((0, 0), (0, npad))).reshape(N_IMG * NPAD)
              for i in (1, 2, 3, 4)]
    perm = jnp.asarray(_PERM_FULL)
    idxb = (topidx.reshape(N_IMG, TOPROWS * 128)
            + (jnp.arange(N_IMG, dtype=jnp.int32) * NPAD)[:, None]
            ).reshape(N_IMG * TOPROWS * 128)
    g0, g1, g2, g3 = _make_gather()(planes[0], planes[1], planes[2],
                                    planes[3], perm, idxb)

    def q(a):
        return a.reshape(N_IMG, TOPROWS, 128)

    props = pl.pallas_call(
        _nms_kernel,
        in_specs=[pl.BlockSpec((N_IMG, TOPROWS, 128), lambda: (0, 0, 0))] * 4,
        out_specs=pl.BlockSpec((N_IMG, 304, 128), lambda: (0, 0, 0)),
        out_shape=jax.ShapeDtypeStruct((N_IMG, 304, 128), jnp.float32),
    )(q(g0), q(g1), q(g2), q(g3))
    return props[:, :POST_NMS, :4]


# speedup vs baseline: 12.2842x; 1.4244x over previous
import functools
import math

import jax
import jax.numpy as jnp
import numpy as np
from jax import lax
from jax.experimental import pallas as pl
from jax.experimental.pallas import tpu as pltpu
from jax.experimental.pallas import tpu_sc as plsc

N_IMG = 2
C = 256
SCALES = [8.0]
RATIOS = [0.5, 1.0, 2.0]
STRIDES = [4, 8, 16, 32]
SHAPES = [(128, 128), (64, 64), (32, 32), (16, 16)]
TILE_H = [32, 32, 32, 16]
PRE_NMS = 6000
POST_NMS = 300
NMS_T = 0.7
NTOT = 3 * sum(h * w for h, w in SHAPES)
NPAD = 65536
ROWS = NPAD // 128
TOPROWS = 48
BIG = 1 << 30

_JS, _KS = [], []
_k = 2
while _k <= NPAD:
    _j = _k // 2
    while _j >= 1:
        _JS.append(_j)
        _KS.append(_k)
        _j //= 2
    _k *= 2
N_STAGES = len(_JS)

_PERM = np.zeros(NPAD, np.int32)
_b = 0
for _h, _w in SHAPES:
    _hw = _h * _w
    _yx = np.arange(_hw)
    for _a in range(3):
        _PERM[_b + _yx * 3 + _a] = _b + _a * _hw + _yx
    _b += 3 * _hw
_PERM[NTOT:] = np.arange(NTOT, NPAD)
_PERM_FULL = np.concatenate(
    [_PERM + _i * NPAD for _i in range(N_IMG)]).astype(np.int32)


def _base_anchors(stride):
    out = []
    c = stride / 2.0
    for s in SCALES:
        for r in RATIOS:
            h = stride * s * math.sqrt(r)
            w = stride * s * math.sqrt(1.0 / r)
            out.append((np.float32(c - h / 2.0), np.float32(c - w / 2.0),
                        np.float32(c + h / 2.0), np.float32(c + w / 2.0)))
    return out


def _head_kernel(H, W, Th, stride, base, idx_base,
                 xb_ref, w9_ref, cb_ref, hw_ref, hb_ref,
                 vs_ref, s_ref, y1_ref, x1_ref, y2_ref, x2_ref, id_ref):
    n = pl.program_id(0)
    t = pl.program_id(1)
    N = Th * W
    lw = W.bit_length() - 1
    p = jax.lax.broadcasted_iota(jnp.int32, (1, N), 1)
    iy = p >> lw
    ix = p & (W - 1)
    mask0 = ix == 0
    maskw = ix == W - 1
    acc = jnp.zeros((C, N), jnp.float32)
    for dy in range(3):
        xc = xb_ref[0, 0, :, dy * W:dy * W + N]
        for dx in range(3):
            if dx == 0:
                xs = jnp.where(mask0, 0.0, pltpu.roll(xc, 1, 1))
            elif dx == 1:
                xs = xc
            else:
                xs = jnp.where(maskw, 0.0, pltpu.roll(xc, N - 1, 1))
            acc = acc + jax.lax.dot(w9_ref[dy * 3 + dx], xs,
                                    preferred_element_type=jnp.float32)
    h1 = jnp.maximum(acc + cb_ref[...], 0.0)
    out16 = jax.lax.dot(hw_ref[...], h1,
                        preferred_element_type=jnp.float32) + hb_ref[...]

    hmax = vs_ref[n, 0].astype(jnp.float32)
    wmax = vs_ref[n, 1].astype(jnp.float32)
    gy = t * Th + iy
    gyf = (gy * stride).astype(jnp.float32)
    gxf = (ix * stride).astype(jnp.float32)
    flat3 = (gy * W + ix) * 3

    for a in range(3):
        s_ref[0, a:a + 1, :] = out16[a:a + 1, :]
        id_ref[0, a:a + 1, :] = idx_base + flat3 + a
        a0, a1, a2, a3 = base[a]
        A0 = a0 + gyf
        A1 = a1 + gxf
        A2 = a2 + gyf
        A3 = a3 + gxf
        ya = (A0 + A2) * 0.5
        xa = (A1 + A3) * 0.5
        ha = A2 - A0
        wa = A3 - A1
        d0 = out16[3 + 4 * a:4 + 4 * a, :]
        d1 = out16[4 + 4 * a:5 + 4 * a, :]
        d2 = out16[5 + 4 * a:6 + 4 * a, :]
        d3 = out16[6 + 4 * a:7 + 4 * a, :]
        yc = ya + d0 * ha
        xc = xa + d1 * wa
        hh = ha * jnp.exp(d2)
        ww = wa * jnp.exp(d3)
        b0 = yc - hh * 0.5
        b1 = xc - ww * 0.5
        b2 = yc + hh * 0.5
        b3 = xc + ww * 0.5
        b0 = jnp.minimum(jnp.maximum(b0, 0.0), hmax)
        b1 = jnp.minimum(jnp.maximum(b1, 0.0), wmax)
        b2 = jnp.minimum(jnp.maximum(b2, 0.0), hmax)
        b3 = jnp.minimum(jnp.maximum(b3, 0.0), wmax)
        y1_ref[0, a:a + 1, :] = b0
        x1_ref[0, a:a + 1, :] = b1
        y2_ref[0, a:a + 1, :] = b2
        x2_ref[0, a:a + 1, :] = b3


def _run_head(x, w9, cb, hw, hb, valid_size, level, idx_base):
    H, W = SHAPES[level]
    Th = TILE_H[level]
    T = H // Th
    stride = STRIDES[level]
    xf = jnp.pad(x, ((0, 0), (0, 0), (1, 1), (0, 0))).reshape(
        N_IMG, C, (H + 2) * W)
    xb = jnp.stack(
        [xf[:, :, t * Th * W:(t * Th + Th + 2) * W] for t in range(T)],
        axis=1)
    HW = H * W
    fs = jax.ShapeDtypeStruct((N_IMG, 3, HW), jnp.float32)
    out_shape = [fs, fs, fs, fs, fs,
                 jax.ShapeDtypeStruct((N_IMG, 3, HW), jnp.int32)]
    obs = pl.BlockSpec((1, 3, Th * W), lambda n, t: (n, 0, t))
    kern = functools.partial(_head_kernel, H, W, Th, stride,
                             _base_anchors(stride), idx_base)
    xspec = pl.BlockSpec((1, 1, C, (Th + 2) * W), lambda n, t: (n, t, 0, 0))
    return pl.pallas_call(
        kern,
        grid=(N_IMG, T),
        in_specs=[
            xspec,
            pl.BlockSpec((9, C, C), lambda n, t: (0, 0, 0)),
            pl.BlockSpec((C, 1), lambda n, t: (0, 0)),
            pl.BlockSpec((16, C), lambda n, t: (0, 0)),
            pl.BlockSpec((16, 1), lambda n, t: (0, 0)),
            pl.BlockSpec(memory_space=pltpu.SMEM),
        ],
        out_specs=[obs] * 6,
        out_shape=out_shape,
    )(xb, w9, cb, hw, hb, valid_size)


def _sort_kernel(js_ref, ks_ref, s_ref, i_ref, o_ref, ksc, isc):
    gio = (jax.lax.broadcasted_iota(jnp.int32, (ROWS, 128), 0) * 128
           + jax.lax.broadcasted_iota(jnp.int32, (ROWS, 128), 1))
    ksc[...] = s_ref[...]
    isc[...] = i_ref[...]

    def stage(s, _):
        j = js_ref[s]
        kstep = ks_ref[s]
        sh1 = j >> 7
        sh2 = j & 127
        m = ((gio & j) == 0)[None]

        def part(x):
            x1r = jnp.where(m, pltpu.roll(x, (ROWS - sh1) & (ROWS - 1), 1),
                            pltpu.roll(x, sh1, 1))
            return jnp.where(m, pltpu.roll(x1r, (128 - sh2) & 127, 2),
                             pltpu.roll(x1r, sh2, 2))

        kk = ksc[...]
        ii = isc[...]
        kp = part(kk)
        ip = part(ii)
        want_first = (m == ((gio & kstep) == 0)[None])
        cur_first = (kk > kp) | ((kk == kp) & (ii < ip))
        tc = want_first == cur_first
        ksc[...] = jnp.where(tc, kk, kp)
        isc[...] = jnp.where(tc, ii, ip)
        return 0

    jax.lax.fori_loop(0, N_STAGES, stage, 0)
    o_ref[...] = isc[:, :TOPROWS, :]


def _make_gather():
    NC, NS = 2, 16
    NW = NC * NS
    B = N_IMG * TOPROWS * 128
    b_per_w = B // NW
    mesh = plsc.VectorSubcoreMesh(core_axis_name="c", subcore_axis_name="s")

    fdt = jax.ShapeDtypeStruct((B,), jnp.float32)

    @functools.partial(
        pl.kernel, mesh=mesh,
        out_type=[fdt, fdt, fdt, fdt],
        scratch_types=[
            pltpu.VMEM((b_per_w,), jnp.int32),
            pltpu.VMEM((b_per_w,), jnp.int32),
            pltpu.VMEM((b_per_w,), jnp.float32),
            pltpu.VMEM((b_per_w,), jnp.float32),
            pltpu.VMEM((b_per_w,), jnp.float32),
            pltpu.VMEM((b_per_w,), jnp.float32),
            pltpu.SemaphoreType.DMA,
        ],
    )
    def gk(c0_hbm, c1_hbm, c2_hbm, c3_hbm, perm_hbm, idx_hbm,
           o0, o1, o2, o3, idx_v, pos_v, b0, b1, b2, b3, sem):
        wid = lax.axis_index("s") * NC + lax.axis_index("c")
        base = wid * b_per_w
        pltpu.sync_copy(idx_hbm.at[pl.ds(base, b_per_w)], idx_v)
        pltpu.async_copy(perm_hbm.at[idx_v], pos_v, sem).wait()
        for src, buf, out in ((c0_hbm, b0, o0), (c1_hbm, b1, o1),
                              (c2_hbm, b2, o2), (c3_hbm, b3, o3)):
            pltpu.async_copy(src.at[pos_v], buf, sem).wait()
            pltpu.sync_copy(buf, out.at[pl.ds(base, b_per_w)])

    return gk


def _nms_kernel(y1_ref, x1_ref, y2_ref, x2_ref, o_ref):
    y1s = y1_ref[...]
    x1s = x1_ref[...]
    y2s = y2_ref[...]
    x2s = x2_ref[...]
    g48 = (jax.lax.broadcasted_iota(jnp.int32, (TOPROWS, 128), 0) * 128
           + jax.lax.broadcasted_iota(jnp.int32, (TOPROWS, 128), 1))[None]
    areas = (y2s - y1s) * (x2s - x1s)
    active0 = jnp.broadcast_to(g48 < PRE_NMS,
                               (N_IMG, TOPROWS, 128)).astype(jnp.int32)
    li = jax.lax.broadcasted_iota(jnp.int32, (1, 1, 128), 2)

    def body(i, active_i):
        active = active_i != 0
        mm = jnp.where(active, g48, BIG)
        mn = jnp.min(mm, axis=(1, 2), keepdims=True)
        valid = mn < BIG
        oh = (g48 == mn).astype(jnp.float32)
        y1c = jnp.sum(y1s * oh, axis=(1, 2), keepdims=True)
        x1c = jnp.sum(x1s * oh, axis=(1, 2), keepdims=True)
        y2c = jnp.sum(y2s * oh, axis=(1, 2), keepdims=True)
        x2c = jnp.sum(x2s * oh, axis=(1, 2), keepdims=True)
        ac = (y2c - y1c) * (x2c - x1c)
        yy1 = jnp.maximum(y1c, y1s)
        xx1 = jnp.maximum(x1c, x1s)
        yy2 = jnp.minimum(y2c, y2s)
        xx2 = jnp.minimum(x2c, x2s)
        inter = jnp.maximum(yy2 - yy1, 0.0) * jnp.maximum(xx2 - xx1, 0.0)
        iou = inter / (ac + areas - inter + 1e-9)
        sup = (iou > NMS_T) & valid
        active = active & (~sup) & (g48 != mn)
        row = (jnp.where(li == 0, y1c, 0.0) + jnp.where(li == 1, x1c, 0.0)
               + jnp.where(li == 2, y2c, 0.0) + jnp.where(li == 3, x2c, 0.0))
        row = jnp.where(valid, row, 0.0)
        o_ref[:, pl.ds(i, 1), :] = row
        return active.astype(jnp.int32)

    jax.lax.fori_loop(0, POST_NMS, body, active0)


def kernel(x0, x1, x2, x3, conv_w, conv_b, obj_w, obj_b, bbx_w, bbx_b,
           valid_size):
    xs = [x0, x1, x2, x3]
    w9 = jnp.transpose(conv_w, (2, 3, 0, 1)).reshape(9, C, C)
    cb = conv_b.reshape(C, 1)
    hw = jnp.concatenate([obj_w[:, :, 0, 0], bbx_w[:, :, 0, 0],
                          jnp.zeros((1, C), jnp.float32)], axis=0)
    hb = jnp.concatenate([obj_b, bbx_b,
                          jnp.zeros((1,), jnp.float32)]).reshape(16, 1)

    parts = [[] for _ in range(6)]
    idx_base = 0
    for level in range(4):
        outs = _run_head(xs[level], w9, cb, hw, hb, valid_size, level,
                         idx_base)
        for p, o in zip(parts, outs):
            p.append(o.reshape(N_IMG, -1))
        idx_base += 3 * SHAPES[level][0] * SHAPES[level][1]

    npad = NPAD - NTOT
    sc = jnp.concatenate(parts[0] + [jnp.full((N_IMG, npad), -jnp.inf,
                                              jnp.float32)], axis=1)
    ids = jnp.concatenate(
        parts[5] + [jnp.broadcast_to(jnp.arange(NTOT, NPAD, dtype=jnp.int32),
                                     (N_IMG, npad))], axis=1)

    def r(a):
        return a.reshape(N_IMG, ROWS, 128)

    js = jnp.asarray(np.array(_JS, np.int32))
    ks = jnp.asarray(np.array(_KS, np.int32))
    topidx = pl.pallas_call(
        _sort_kernel,
        in_specs=[pl.BlockSpec(memory_space=pltpu.SMEM)] * 2
        + [pl.BlockSpec((N_IMG, ROWS, 128), lambda: (0, 0, 0))] * 2,
        out_specs=pl.BlockSpec((N_IMG, TOPROWS, 128), lambda: (0, 0, 0)),
        out_shape=jax.ShapeDtypeStruct((N_IMG, TOPROWS, 128), jnp.int32),
        scratch_shapes=[pltpu.VMEM((N_IMG, ROWS, 128), jnp.float32),
                        pltpu.VMEM((N_IMG, ROWS, 128), jnp.int32)],
    )(js, ks, r(sc), r(ids))

    planes = [jnp.pad(jnp.concatenate(parts[i], axis=1),
                      ((0, 0), (0, npad))).reshape(N_IMG * NPAD)
              for i in (1, 2, 3, 4)]
    perm = jnp.asarray(_PERM_FULL)
    idxb = (topidx.reshape(N_IMG, TOPROWS * 128)
            + (jnp.arange(N_IMG, dtype=jnp.int32) * NPAD)[:, None]
            ).reshape(N_IMG * TOPROWS * 128)
    g0, g1, g2, g3 = _make_gather()(planes[0], planes[1], planes[2],
                                    planes[3], perm, idxb)

    def q(a):
        return a.reshape(N_IMG, TOPROWS, 128)

    props = pl.pallas_call(
        _nms_kernel,
        in_specs=[pl.BlockSpec((N_IMG, TOPROWS, 128), lambda: (0, 0, 0))] * 4,
        out_specs=pl.BlockSpec((N_IMG, 304, 128), lambda: (0, 0, 0)),
        out_shape=jax.ShapeDtypeStruct((N_IMG, 304, 128), jnp.float32),
    )(q(g0), q(g1), q(g2), q(g3))
    return props[:, :POST_NMS, :4]


# speedup vs baseline: 14.7305x; 1.1991x over previous
import functools
import math

import jax
import jax.numpy as jnp
import numpy as np
from jax import lax
from jax.experimental import pallas as pl
from jax.experimental.pallas import tpu as pltpu
from jax.experimental.pallas import tpu_sc as plsc

N_IMG = 2
C = 256
SCALES = [8.0]
RATIOS = [0.5, 1.0, 2.0]
STRIDES = [4, 8, 16, 32]
SHAPES = [(128, 128), (64, 64), (32, 32), (16, 16)]
TILE_H = [32, 32, 32, 16]
PRE_NMS = 6000
POST_NMS = 300
NMS_T = 0.7
NTOT = 3 * sum(h * w for h, w in SHAPES)
NPAD = 65536
ROWS = NPAD // 128
TOPROWS = 48
BIG = 1 << 30

_JS, _KS = [], []
_k = 2
while _k <= NPAD:
    _j = _k // 2
    while _j >= 1:
        _JS.append(_j)
        _KS.append(_k)
        _j //= 2
    _k *= 2
N_STAGES = len(_JS)
_RUNS = []
for _s, _jv in enumerate(_JS):
    _kind = "lane" if _jv < 128 else "row"
    if _RUNS and _RUNS[-1][0] == _kind:
        _RUNS[-1][2] = _s + 1
    else:
        _RUNS.append([_kind, _s, _s + 1])

_PERM = np.zeros(NPAD, np.int32)
_b = 0
for _h, _w in SHAPES:
    _hw = _h * _w
    _yx = np.arange(_hw)
    for _a in range(3):
        _PERM[_b + _yx * 3 + _a] = _b + _a * _hw + _yx
    _b += 3 * _hw
_PERM[NTOT:] = np.arange(NTOT, NPAD)
_PERM_FULL = np.concatenate(
    [_PERM + _i * NPAD for _i in range(N_IMG)]).astype(np.int32)


def _base_anchors(stride):
    out = []
    c = stride / 2.0
    for s in SCALES:
        for r in RATIOS:
            h = stride * s * math.sqrt(r)
            w = stride * s * math.sqrt(1.0 / r)
            out.append((np.float32(c - h / 2.0), np.float32(c - w / 2.0),
                        np.float32(c + h / 2.0), np.float32(c + w / 2.0)))
    return out


def _head_kernel(H, W, Th, stride, base, idx_base,
                 xb_ref, w9_ref, cb_ref, hw_ref, hb_ref,
                 vs_ref, s_ref, y1_ref, x1_ref, y2_ref, x2_ref, id_ref):
    n = pl.program_id(0)
    t = pl.program_id(1)
    N = Th * W
    lw = W.bit_length() - 1
    p = jax.lax.broadcasted_iota(jnp.int32, (1, N), 1)
    iy = p >> lw
    ix = p & (W - 1)
    mask0 = ix == 0
    maskw = ix == W - 1
    acc = jnp.zeros((C, N), jnp.float32)
    for dy in range(3):
        xc = xb_ref[0, 0, :, dy * W:dy * W + N]
        for dx in range(3):
            if dx == 0:
                xs = jnp.where(mask0, 0.0, pltpu.roll(xc, 1, 1))
            elif dx == 1:
                xs = xc
            else:
                xs = jnp.where(maskw, 0.0, pltpu.roll(xc, N - 1, 1))
            acc = acc + jax.lax.dot(w9_ref[dy * 3 + dx], xs,
                                    preferred_element_type=jnp.float32)
    h1 = jnp.maximum(acc + cb_ref[...], 0.0)
    out16 = jax.lax.dot(hw_ref[...], h1,
                        preferred_element_type=jnp.float32) + hb_ref[...]

    hmax = vs_ref[n, 0].astype(jnp.float32)
    wmax = vs_ref[n, 1].astype(jnp.float32)
    gy = t * Th + iy
    gyf = (gy * stride).astype(jnp.float32)
    gxf = (ix * stride).astype(jnp.float32)
    flat3 = (gy * W + ix) * 3

    for a in range(3):
        s_ref[0, a:a + 1, :] = out16[a:a + 1, :]
        id_ref[0, a:a + 1, :] = idx_base + flat3 + a
        a0, a1, a2, a3 = base[a]
        A0 = a0 + gyf
        A1 = a1 + gxf
        A2 = a2 + gyf
        A3 = a3 + gxf
        ya = (A0 + A2) * 0.5
        xa = (A1 + A3) * 0.5
        ha = A2 - A0
        wa = A3 - A1
        d0 = out16[3 + 4 * a:4 + 4 * a, :]
        d1 = out16[4 + 4 * a:5 + 4 * a, :]
        d2 = out16[5 + 4 * a:6 + 4 * a, :]
        d3 = out16[6 + 4 * a:7 + 4 * a, :]
        yc = ya + d0 * ha
        xc = xa + d1 * wa
        hh = ha * jnp.exp(d2)
        ww = wa * jnp.exp(d3)
        b0 = yc - hh * 0.5
        b1 = xc - ww * 0.5
        b2 = yc + hh * 0.5
        b3 = xc + ww * 0.5
        b0 = jnp.minimum(jnp.maximum(b0, 0.0), hmax)
        b1 = jnp.minimum(jnp.maximum(b1, 0.0), wmax)
        b2 = jnp.minimum(jnp.maximum(b2, 0.0), hmax)
        b3 = jnp.minimum(jnp.maximum(b3, 0.0), wmax)
        y1_ref[0, a:a + 1, :] = b0
        x1_ref[0, a:a + 1, :] = b1
        y2_ref[0, a:a + 1, :] = b2
        x2_ref[0, a:a + 1, :] = b3


def _run_head(x, w9, cb, hw, hb, valid_size, level, idx_base):
    H, W = SHAPES[level]
    Th = TILE_H[level]
    T = H // Th
    stride = STRIDES[level]
    xf = jnp.pad(x, ((0, 0), (0, 0), (1, 1), (0, 0))).reshape(
        N_IMG, C, (H + 2) * W)
    xb = jnp.stack(
        [xf[:, :, t * Th * W:(t * Th + Th + 2) * W] for t in range(T)],
        axis=1)
    HW = H * W
    fs = jax.ShapeDtypeStruct((N_IMG, 3, HW), jnp.float32)
    out_shape = [fs, fs, fs, fs, fs,
                 jax.ShapeDtypeStruct((N_IMG, 3, HW), jnp.int32)]
    obs = pl.BlockSpec((1, 3, Th * W), lambda n, t: (n, 0, t))
    kern = functools.partial(_head_kernel, H, W, Th, stride,
                             _base_anchors(stride), idx_base)
    xspec = pl.BlockSpec((1, 1, C, (Th + 2) * W), lambda n, t: (n, t, 0, 0))
    return pl.pallas_call(
        kern,
        grid=(N_IMG, T),
        in_specs=[
            xspec,
            pl.BlockSpec((9, C, C), lambda n, t: (0, 0, 0)),
            pl.BlockSpec((C, 1), lambda n, t: (0, 0)),
            pl.BlockSpec((16, C), lambda n, t: (0, 0)),
            pl.BlockSpec((16, 1), lambda n, t: (0, 0)),
            pl.BlockSpec(memory_space=pltpu.SMEM),
        ],
        out_specs=[obs] * 6,
        out_shape=out_shape,
    )(xb, w9, cb, hw, hb, valid_size)


def _sort_kernel(js_ref, ks_ref, s_ref, i_ref, o_ref, ksc, isc):
    gio = (jax.lax.broadcasted_iota(jnp.int32, (ROWS, 128), 0) * 128
           + jax.lax.broadcasted_iota(jnp.int32, (ROWS, 128), 1))
    ksc[...] = s_ref[...]
    isc[...] = i_ref[...]

    def exchange(s, m, part):
        kstep = ks_ref[s]
        kk = ksc[...]
        ii = isc[...]
        kp = part(kk)
        ip = part(ii)
        want_first = (m == ((gio & kstep) == 0)[None])
        cur_first = (kk > kp) | ((kk == kp) & (ii < ip))
        tc = want_first == cur_first
        ksc[...] = jnp.where(tc, kk, kp)
        isc[...] = jnp.where(tc, ii, ip)
        return 0

    def lane_stage(s, _):
        j = js_ref[s]
        m = ((gio & j) == 0)[None]

        def part(x):
            return jnp.where(m, pltpu.roll(x, (128 - j) & 127, 2),
                             pltpu.roll(x, j, 2))

        return exchange(s, m, part)

    def row_stage(s, _):
        j = js_ref[s]
        sh1 = j >> 7
        m = ((gio & j) == 0)[None]

        def part(x):
            return jnp.where(m, pltpu.roll(x, (ROWS - sh1) & (ROWS - 1), 1),
                             pltpu.roll(x, sh1, 1))

        return exchange(s, m, part)

    for _kind, _a, _b in _RUNS:
        jax.lax.fori_loop(_a, _b,
                          lane_stage if _kind == "lane" else row_stage, 0)
    o_ref[...] = isc[:, :TOPROWS, :]


def _make_gather():
    NC, NS = 2, 16
    NW = NC * NS
    B = N_IMG * TOPROWS * 128
    b_per_w = B // NW
    mesh = plsc.VectorSubcoreMesh(core_axis_name="c", subcore_axis_name="s")

    fdt = jax.ShapeDtypeStruct((B,), jnp.float32)

    @functools.partial(
        pl.kernel, mesh=mesh,
        out_type=[fdt, fdt, fdt, fdt],
        scratch_types=[
            pltpu.VMEM((b_per_w,), jnp.int32),
            pltpu.VMEM((b_per_w,), jnp.int32),
            pltpu.VMEM((b_per_w,), jnp.float32),
            pltpu.VMEM((b_per_w,), jnp.float32),
            pltpu.VMEM((b_per_w,), jnp.float32),
            pltpu.VMEM((b_per_w,), jnp.float32),
            pltpu.SemaphoreType.DMA,
        ],
    )
    def gk(c0_hbm, c1_hbm, c2_hbm, c3_hbm, perm_hbm, idx_hbm,
           o0, o1, o2, o3, idx_v, pos_v, b0, b1, b2, b3, sem):
        wid = lax.axis_index("s") * NC + lax.axis_index("c")
        base = wid * b_per_w
        pltpu.sync_copy(idx_hbm.at[pl.ds(base, b_per_w)], idx_v)
        pltpu.async_copy(perm_hbm.at[idx_v], pos_v, sem).wait()
        for src, buf, out in ((c0_hbm, b0, o0), (c1_hbm, b1, o1),
                              (c2_hbm, b2, o2), (c3_hbm, b3, o3)):
            pltpu.async_copy(src.at[pos_v], buf, sem).wait()
            pltpu.sync_copy(buf, out.at[pl.ds(base, b_per_w)])

    return gk


def _nms_kernel(y1_ref, x1_ref, y2_ref, x2_ref, o_ref):
    y1s = y1_ref[...]
    x1s = x1_ref[...]
    y2s = y2_ref[...]
    x2s = x2_ref[...]
    g48 = (jax.lax.broadcasted_iota(jnp.int32, (TOPROWS, 128), 0) * 128
           + jax.lax.broadcasted_iota(jnp.int32, (TOPROWS, 128), 1))[None]
    areas = (y2s - y1s) * (x2s - x1s)
    active0 = jnp.broadcast_to(g48 < PRE_NMS,
                               (N_IMG, TOPROWS, 128)).astype(jnp.int32)
    li = jax.lax.broadcasted_iota(jnp.int32, (1, 1, 128), 2)

    def body(i, active_i):
        active = active_i != 0
        mm = jnp.where(active, g48, BIG)
        mn = jnp.min(mm, axis=(1, 2), keepdims=True)
        valid = mn < BIG
        oh = (g48 == mn).astype(jnp.float32)
        y1c = jnp.sum(y1s * oh, axis=(1, 2), keepdims=True)
        x1c = jnp.sum(x1s * oh, axis=(1, 2), keepdims=True)
        y2c = jnp.sum(y2s * oh, axis=(1, 2), keepdims=True)
        x2c = jnp.sum(x2s * oh, axis=(1, 2), keepdims=True)
        ac = (y2c - y1c) * (x2c - x1c)
        yy1 = jnp.maximum(y1c, y1s)
        xx1 = jnp.maximum(x1c, x1s)
        yy2 = jnp.minimum(y2c, y2s)
        xx2 = jnp.minimum(x2c, x2s)
        inter = jnp.maximum(yy2 - yy1, 0.0) * jnp.maximum(xx2 - xx1, 0.0)
        iou = inter / (ac + areas - inter + 1e-9)
        sup = (iou > NMS_T) & valid
        active = active & (~sup) & (g48 != mn)
        row = (jnp.where(li == 0, y1c, 0.0) + jnp.where(li == 1, x1c, 0.0)
               + jnp.where(li == 2, y2c, 0.0) + jnp.where(li == 3, x2c, 0.0))
        row = jnp.where(valid, row, 0.0)
        o_ref[:, pl.ds(i, 1), :] = row
        return active.astype(jnp.int32)

    jax.lax.fori_loop(0, POST_NMS, body, active0)


def kernel(x0, x1, x2, x3, conv_w, conv_b, obj_w, obj_b, bbx_w, bbx_b,
           valid_size):
    xs = [x0, x1, x2, x3]
    w9 = jnp.transpose(conv_w, (2, 3, 0, 1)).reshape(9, C, C)
    cb = conv_b.reshape(C, 1)
    hw = jnp.concatenate([obj_w[:, :, 0, 0], bbx_w[:, :, 0, 0],
                          jnp.zeros((1, C), jnp.float32)], axis=0)
    hb = jnp.concatenate([obj_b, bbx_b,
                          jnp.zeros((1,), jnp.float32)]).reshape(16, 1)

    parts = [[] for _ in range(6)]
    idx_base = 0
    for level in range(4):
        outs = _run_head(xs[level], w9, cb, hw, hb, valid_size, level,
                         idx_base)
        for p, o in zip(parts, outs):
            p.append(o.reshape(N_IMG, -1))
        idx_base += 3 * SHAPES[level][0] * SHAPES[level][1]

    npad = NPAD - NTOT
    sc = jnp.concatenate(parts[0] + [jnp.full((N_IMG, npad), -jnp.inf,
                                              jnp.float32)], axis=1)
    ids = jnp.concatenate(
        parts[5] + [jnp.broadcast_to(jnp.arange(NTOT, NPAD, dtype=jnp.int32),
                                     (N_IMG, npad))], axis=1)

    def r(a):
        return a.reshape(N_IMG, ROWS, 128)

    js = jnp.asarray(np.array(_JS, np.int32))
    ks = jnp.asarray(np.array(_KS, np.int32))
    topidx = pl.pallas_call(
        _sort_kernel,
        in_specs=[pl.BlockSpec(memory_space=pltpu.SMEM)] * 2
        + [pl.BlockSpec((N_IMG, ROWS, 128), lambda: (0, 0, 0))] * 2,
        out_specs=pl.BlockSpec((N_IMG, TOPROWS, 128), lambda: (0, 0, 0)),
        out_shape=jax.ShapeDtypeStruct((N_IMG, TOPROWS, 128), jnp.int32),
        scratch_shapes=[pltpu.VMEM((N_IMG, ROWS, 128), jnp.float32),
                        pltpu.VMEM((N_IMG, ROWS, 128), jnp.int32)],
    )(js, ks, r(sc), r(ids))

    planes = [jnp.pad(jnp.concatenate(parts[i], axis=1),
                      ((0, 0), (0, npad))).reshape(N_IMG * NPAD)
              for i in (1, 2, 3, 4)]
    perm = jnp.asarray(_PERM_FULL)
    idxb = (topidx.reshape(N_IMG, TOPROWS * 128)
            + (jnp.arange(N_IMG, dtype=jnp.int32) * NPAD)[:, None]
            ).reshape(N_IMG * TOPROWS * 128)
    g0, g1, g2, g3 = _make_gather()(planes[0], planes[1], planes[2],
                                    planes[3], perm, idxb)

    def q(a):
        return a.reshape(N_IMG, TOPROWS, 128)

    props = pl.pallas_call(
        _nms_kernel,
        in_specs=[pl.BlockSpec((N_IMG, TOPROWS, 128), lambda: (0, 0, 0))] * 4,
        out_specs=pl.BlockSpec((N_IMG, 304, 128), lambda: (0, 0, 0)),
        out_shape=jax.ShapeDtypeStruct((N_IMG, 304, 128), jnp.float32),
    )(q(g0), q(g1), q(g2), q(g3))
    return props[:, :POST_NMS, :4]
